# baseline (speedup 1.0000x reference)
import jax
import jax.numpy as jnp
from jax.experimental import pallas as pl
from jax.experimental.pallas import tpu as pltpu

DEPTH_ = 1000
ROWS_PER_BLOCK = 32
NBUF = 4


def _onehot_body(idx_ref, out_ref, scratch, sems):
    i = pl.program_id(0)
    n = pl.num_programs(0)
    slot = jax.lax.rem(i, NBUF)
    r = ROWS_PER_BLOCK

    @pl.when(i >= NBUF)
    def _wait_slot():
        pltpu.make_async_copy(
            scratch.at[slot], out_ref.at[pl.ds(0, r)], sems.at[slot]
        ).wait()

    iota = jax.lax.broadcasted_iota(jnp.int32, (r, 26, DEPTH_), 2)
    scratch[slot] = (iota == idx_ref[...]).astype(jnp.float32)

    pltpu.make_async_copy(
        scratch.at[slot], out_ref.at[pl.ds(i * r, r)], sems.at[slot]
    ).start()

    @pl.when(i == n - 1)
    def _drain():
        for j in range(NBUF):
            pltpu.make_async_copy(
                scratch.at[j], out_ref.at[pl.ds(0, r)], sems.at[j]
            ).wait()


def kernel(indices):
    b, f = indices.shape
    idx3 = indices.astype(jnp.int32)[..., None]
    out = pl.pallas_call(
        _onehot_body,
        grid=(b // ROWS_PER_BLOCK,),
        in_specs=[pl.BlockSpec((ROWS_PER_BLOCK, f, 1), lambda i: (i, 0, 0))],
        out_specs=pl.BlockSpec(memory_space=pl.ANY),
        out_shape=jax.ShapeDtypeStruct((b, f, DEPTH_), jnp.float32),
        scratch_shapes=[
            pltpu.VMEM((NBUF, ROWS_PER_BLOCK, f, DEPTH_), jnp.float32),
            pltpu.SemaphoreType.DMA((NBUF,)),
        ],
    )(idx3)
    return out


# speedup vs baseline: 1.0602x; 1.0602x over previous
import functools

import jax
import jax.numpy as jnp
from jax import lax
from jax.experimental import pallas as pl
from jax.experimental.pallas import tpu as pltpu
from jax.experimental.pallas import tpu_sc as plsc

B_ = 4096
F_ = 26
DEPTH_ = 1000
NC_ = 2
NS_ = 16
NW_ = NC_ * NS_
EPW_ = B_ // NW_
EPB_ = 1
NBUF_ = 2
NBATCH_ = EPW_ // EPB_

_mesh = plsc.VectorSubcoreMesh(core_axis_name="c", subcore_axis_name="s")


@functools.partial(
    pl.kernel,
    mesh=_mesh,
    out_type=jax.ShapeDtypeStruct((B_, F_, DEPTH_), jnp.float32),
    scratch_types=[
        pltpu.VMEM((EPW_ * F_,), jnp.int32),
        pltpu.VMEM((EPB_, F_, DEPTH_), jnp.float32),
        pltpu.VMEM((EPB_, F_, DEPTH_), jnp.float32),
        pltpu.SemaphoreType.DMA((NBUF_,)),
    ],
    compiler_params=pltpu.CompilerParams(needs_layout_passes=False),
)
def _sc_onehot(idx_hbm, out_hbm, idx_v, buf0, buf1, sems):
    wid = lax.axis_index("s") * NC_ + lax.axis_index("c")
    ebase = wid * EPW_
    bufs = [buf0, buf1]

    iota = lax.iota(jnp.int32, 16)
    ones = jnp.full((16,), 1.0, jnp.float32)
    zeros = jnp.zeros((16,), jnp.float32)

    pltpu.sync_copy(idx_hbm.at[pl.ds(ebase * F_, EPW_ * F_)], idx_v)

    for buf in bufs:
        def zero_body(k, _, buf=buf):
            e = k // F_
            f = k % F_
            d0 = jnp.full((16,), e, jnp.int32)
            d1 = jnp.full((16,), f, jnp.int32)
            for c in range(63):
                d2 = c * 16 + iota
                plsc.store_scatter(buf, [d0, d1, d2], zeros, mask=d2 < DEPTH_)
            return 0
        lax.fori_loop(0, EPB_ * F_, zero_body, 0)

    def scatter_elem(buf, gb, e, vals16):
        row = gb * EPB_ + e
        d0 = jnp.full((16,), e, jnp.int32)
        for go in (0, F_ - 16):
            off = row * F_ + go
            vals = idx_v[pl.ds(off, 16)]
            d1 = go + iota
            plsc.store_scatter(buf, [d0, d1, vals], vals16)

    def batch_body(gb, _):
        q = lax.rem(gb, NBUF_)
        for qq in range(NBUF_):
            @pl.when(q == qq)
            def _do(qq=qq):
                buf = bufs[qq]

                @pl.when(gb >= NBUF_)
                def _recycle():
                    pltpu.make_async_copy(
                        buf, out_hbm.at[pl.ds(ebase, EPB_)], sems.at[qq]
                    ).wait()
                    for e in range(EPB_):
                        scatter_elem(buf, gb - NBUF_, e, zeros)

                for e in range(EPB_):
                    scatter_elem(buf, gb, e, ones)

                pltpu.async_copy(
                    buf,
                    out_hbm.at[pl.ds(ebase + gb * EPB_, EPB_)],
                    sems.at[qq],
                )
        return 0

    lax.fori_loop(0, NBATCH_, batch_body, 0)

    for qq in range(NBUF_):
        pltpu.make_async_copy(
            bufs[qq], out_hbm.at[pl.ds(ebase, EPB_)], sems.at[qq]
        ).wait()


def kernel(indices):
    idx_flat = indices.reshape(-1).astype(jnp.int32)
    return _sc_onehot(idx_flat)


# speedup vs baseline: 4.9895x; 4.7063x over previous
import jax
import jax.numpy as jnp
from jax.experimental import pallas as pl
from jax.experimental.pallas import tpu as pltpu

B_ = 4096
F_ = 26
DEPTH_ = 1000
D_BLK = 8


def _onehot_t_body(idx_ref, out_ref):
    i = pl.program_id(0)
    iota = jax.lax.broadcasted_iota(jnp.int32, (F_, D_BLK, B_), 1) + i * D_BLK
    out_ref[...] = (iota == idx_ref[...]).astype(jnp.float32)


def kernel(indices):
    idx_t = jnp.transpose(indices.astype(jnp.int32))[:, None, :]
    out_t = pl.pallas_call(
        _onehot_t_body,
        grid=(DEPTH_ // D_BLK,),
        in_specs=[pl.BlockSpec((F_, 1, B_), lambda i: (0, 0, 0))],
        out_specs=pl.BlockSpec((F_, D_BLK, B_), lambda i: (0, i, 0)),
        out_shape=jax.ShapeDtypeStruct((F_, DEPTH_, B_), jnp.float32),
    )(idx_t)
    return jnp.transpose(out_t, (2, 0, 1))


# speedup vs baseline: 4.9901x; 1.0001x over previous
import jax
import jax.numpy as jnp
from jax.experimental import pallas as pl
from jax.experimental.pallas import tpu as pltpu

B_ = 4096
F_ = 26
DEPTH_ = 1000
D_BLK = 8


def _onehot_t_body(idx_hbm, out_ref, idx_v):
    i = pl.program_id(0)

    @pl.when(i == 0)
    def _stage_idx():
        pltpu.sync_copy(idx_hbm, idx_v)

    iota = jax.lax.broadcasted_iota(jnp.int32, (F_, D_BLK, B_), 1) + i * D_BLK
    out_ref[...] = (iota == idx_v[...]).astype(jnp.float32)


def kernel(indices):
    idx_t = jnp.transpose(indices.astype(jnp.int32))[:, None, :]
    out_t = pl.pallas_call(
        _onehot_t_body,
        grid=(DEPTH_ // D_BLK,),
        in_specs=[pl.BlockSpec(memory_space=pl.ANY)],
        out_specs=pl.BlockSpec((F_, D_BLK, B_), lambda i: (0, i, 0)),
        out_shape=jax.ShapeDtypeStruct((F_, DEPTH_, B_), jnp.float32),
        scratch_shapes=[pltpu.VMEM((F_, 1, B_), jnp.int32)],
    )(idx_t)
    return jnp.transpose(out_t, (2, 0, 1))


# speedup vs baseline: 5.1267x; 1.0274x over previous
import jax
import jax.numpy as jnp
from jax.experimental import pallas as pl
from jax.experimental.pallas import tpu as pltpu

B_ = 4096
F_ = 26
DEPTH_ = 1000
D_BLK = 40


def _onehot_t_body(idx_hbm, out_ref, idx_v):
    i = pl.program_id(0)

    @pl.when(i == 0)
    def _stage_idx():
        pltpu.sync_copy(idx_hbm, idx_v)

    iota = jax.lax.broadcasted_iota(jnp.int32, (F_, D_BLK, B_), 1) + i * D_BLK
    out_ref[...] = (iota == idx_v[...]).astype(jnp.float32)


def kernel(indices):
    idx_t = jnp.transpose(indices.astype(jnp.int32))[:, None, :]
    out_t = pl.pallas_call(
        _onehot_t_body,
        grid=(DEPTH_ // D_BLK,),
        in_specs=[pl.BlockSpec(memory_space=pl.ANY)],
        out_specs=pl.BlockSpec((F_, D_BLK, B_), lambda i: (0, i, 0)),
        out_shape=jax.ShapeDtypeStruct((F_, DEPTH_, B_), jnp.float32),
        scratch_shapes=[pltpu.VMEM((F_, 1, B_), jnp.int32)],
    )(idx_t)
    return jnp.transpose(out_t, (2, 0, 1))
